# baseline (speedup 1.0000x reference)
import functools

import jax
import jax.numpy as jnp
from jax import lax
from jax.experimental import pallas as pl
from jax.experimental.pallas import tpu as pltpu
from jax.experimental.pallas import tpu_sc as plsc

NN = 10000
DD = 128
HH = 128
GG = 16

NC = 2
NS = 16
NW = NC * NS

CHUNK = 80
N_PAD = 10240
TRASH = NN
ROW_BLK = 1024
ROWS_PER_TILE = N_PAD // NS


HW = 8


def _sc_hist(e_pad):
  per_tile = e_pad // NW
  n_chunks = per_tile // CHUNK
  mesh = plsc.VectorSubcoreMesh(core_axis_name="c", subcore_axis_name="s")

  n_groups = n_chunks // 5

  @functools.partial(
      pl.kernel,
      mesh=mesh,
      out_type=jax.ShapeDtypeStruct((NC, N_PAD, HW), jnp.float32),
      compiler_params=pltpu.CompilerParams(use_tc_tiling_on_sc=False),
      scratch_types=[
          pltpu.VMEM((n_chunks, CHUNK), jnp.int32),
          pltpu.VMEM((CHUNK, HW), jnp.float32),
          pltpu.VMEM((ROWS_PER_TILE, HW), jnp.float32),
          pltpu.VMEM_SHARED((N_PAD, HW), jnp.float32),
          pltpu.SemaphoreType.DMA((5,)),
      ],
  )
  def hist(dst_hbm, ones_hbm, zeros_hbm, out_hbm,
           dsti_v, ones_v, big_v, acc_sh, ssem):
    c = lax.axis_index("c")
    s = lax.axis_index("s")
    wid = c * NS + s
    row0 = s * ROWS_PER_TILE
    pltpu.sync_copy(zeros_hbm, big_v)
    pltpu.sync_copy(big_v, acc_sh.at[pl.ds(row0, ROWS_PER_TILE)])
    pltpu.sync_copy(ones_hbm, ones_v)
    pltpu.sync_copy(dst_hbm.at[wid], dsti_v)
    plsc.subcore_barrier()

    def scatter(j, b):
      return pltpu.make_async_copy(ones_v, acc_sh.at[dsti_v.at[j]],
                                   ssem.at[b])

    def grp(g, carry):
      for b in range(5):
        @pl.when(g > 0)
        def _():
          scatter(0, b).wait()
        scatter(g * 5 + b, b).start(add=True)
      return carry

    lax.fori_loop(0, n_groups, grp, 0)
    for b in range(5):
      scatter(0, b).wait()
    plsc.subcore_barrier()
    pltpu.sync_copy(acc_sh.at[pl.ds(row0, ROWS_PER_TILE)], big_v)
    pltpu.sync_copy(big_v, out_hbm.at[c, pl.ds(row0, ROWS_PER_TILE)])

  return hist


SPLIT = 2
FS = HH // SPLIT
ZCH = ROWS_PER_TILE // 4


NBUF = 5


def _sc_conv(e_pad):
  per_tile = e_pad // NW
  n_chunks = per_tile // CHUNK
  n_groups = n_chunks // NBUF
  mesh = plsc.VectorSubcoreMesh(core_axis_name="c", subcore_axis_name="s")

  @functools.partial(
      pl.kernel,
      mesh=mesh,
      out_type=[jax.ShapeDtypeStruct((NC, N_PAD, FS), jnp.float32)
                for _ in range(SPLIT)],
      compiler_params=pltpu.CompilerParams(use_tc_tiling_on_sc=False),
      scratch_types=[
          pltpu.VMEM((n_chunks, CHUNK), jnp.int32),
          pltpu.VMEM((n_chunks, CHUNK), jnp.int32),
          [pltpu.VMEM((CHUNK, FS), jnp.float32) for _ in range(NBUF)],
          pltpu.VMEM((ZCH, FS), jnp.float32),
          pltpu.VMEM((ZCH, FS), jnp.float32),
          pltpu.VMEM_SHARED((N_PAD, FS), jnp.float32),
          pltpu.SemaphoreType.DMA((NBUF,)),
          pltpu.SemaphoreType.DMA((NBUF,)),
      ],
  )
  def conv(y0, y1, src_hbm, dst_hbm, out0_hbm, out1_hbm,
           srci_v, dsti_v, rows, zero_v, rd_v, acc_sh, gsem, ssem):
    c = lax.axis_index("c")
    s = lax.axis_index("s")
    wid = c * NS + s
    row0 = s * ROWS_PER_TILE

    pltpu.sync_copy(src_hbm.at[wid], srci_v)
    pltpu.sync_copy(dst_hbm.at[wid], dsti_v)

    def zbody(r, carry):
      z = jnp.zeros((16,), jnp.float32)
      for g in range(FS // 16):
        zero_v[r, pl.ds(g * 16, 16)] = z
      return carry

    lax.fori_loop(0, ZCH, zbody, 0)

    def gather(j, b, y_hbm):
      return pltpu.make_async_copy(y_hbm.at[srci_v.at[j]], rows[b],
                                   gsem.at[b])

    def scatter(j, b):
      return pltpu.make_async_copy(rows[b], acc_sh.at[dsti_v.at[j]],
                                   ssem.at[b])

    for y_hbm, out_hbm in ((y0, out0_hbm), (y1, out1_hbm)):
      for q in range(ROWS_PER_TILE // ZCH):
        pltpu.sync_copy(zero_v, acc_sh.at[pl.ds(row0 + q * ZCH, ZCH)])
      plsc.subcore_barrier()

      for b in range(NBUF):
        gather(b, b, y_hbm).start()

      def grp(g, carry):
        for b in range(NBUF):
          j = g * NBUF + b
          gather(j, b, y_hbm).wait()
          desc = scatter(j, b)
          desc.start(add=True)

        @pl.when(g < n_groups - 1)
        def _():
          for b in range(NBUF):
            j = (g + 1) * NBUF + b
            scatter(j - NBUF, b).wait()
            gather(j, b, y_hbm).start()

        return carry

      lax.fori_loop(0, n_groups, grp, 0)
      for b in range(NBUF):
        scatter(0, b).wait()

      plsc.subcore_barrier()
      for q in range(ROWS_PER_TILE // ZCH):
        pltpu.sync_copy(acc_sh.at[pl.ds(row0 + q * ZCH, ZCH)], rd_v)
        pltpu.sync_copy(rd_v, out_hbm.at[c, pl.ds(row0 + q * ZCH, ZCH)])

  return conv


_GRID = N_PAD // ROW_BLK


def _tc_prep():
  def body(x_ref, w_ref, degp_ref, xw_ref, y0_ref, y1_ref, dinv_ref,
           xsum_ref, hc_ref, hcl_ref):
    i = pl.program_id(0)
    dinv = lax.rsqrt(degp_ref[0, :, 0:1] + degp_ref[1, :, 0:1] + 2.0)
    xw = jnp.dot(x_ref[...], w_ref[...], preferred_element_type=jnp.float32)
    xw_ref[...] = xw
    y = xw * dinv
    y0_ref[...] = y[:, :FS]
    y1_ref[...] = y[:, FS:]
    dinv_ref[...] = dinv

    @pl.when(i == 0)
    def _():
      xsum_ref[...] = jnp.zeros_like(xsum_ref)

    xsum_ref[...] += jnp.sum(x_ref[...], axis=0, keepdims=True)

    @pl.when(i == _GRID - 1)
    def _():
      gs = jnp.sum(xsum_ref[...])
      hc_ref[...] = xsum_ref[...] / gs
      hcl_ref[...] = jnp.full((1, 1), 0.0) + jnp.log(gs)

  return pl.pallas_call(
      body,
      grid=(_GRID,),
      in_specs=[
          pl.BlockSpec((ROW_BLK, DD), lambda i: (i, 0)),
          pl.BlockSpec((DD, HH), lambda i: (0, 0)),
          pl.BlockSpec((NC, ROW_BLK, HW), lambda i: (0, i, 0)),
      ],
      out_specs=[
          pl.BlockSpec((ROW_BLK, HH), lambda i: (i, 0)),
          pl.BlockSpec((ROW_BLK, FS), lambda i: (i, 0)),
          pl.BlockSpec((ROW_BLK, FS), lambda i: (i, 0)),
          pl.BlockSpec((ROW_BLK, 1), lambda i: (i, 0)),
          pl.BlockSpec((1, DD), lambda i: (0, 0)),
          pl.BlockSpec((1, DD), lambda i: (0, 0)),
          pl.BlockSpec((1, 1), lambda i: (0, 0)),
      ],
      out_shape=[
          jax.ShapeDtypeStruct((N_PAD, HH), jnp.float32),
          jax.ShapeDtypeStruct((N_PAD, FS), jnp.float32),
          jax.ShapeDtypeStruct((N_PAD, FS), jnp.float32),
          jax.ShapeDtypeStruct((N_PAD, 1), jnp.float32),
          jax.ShapeDtypeStruct((1, DD), jnp.float32),
          jax.ShapeDtypeStruct((1, DD), jnp.float32),
          jax.ShapeDtypeStruct((1, 1), jnp.float32),
      ],
  )


def _tc_mid():
  def body(acca_ref, accb_ref, xw_ref, dinv_ref, b_ref, w_ref,
           xwn_ref, y0_ref, y1_ref):
    dv = dinv_ref[...]
    acc = jnp.concatenate([acca_ref[0] + acca_ref[1],
                           accb_ref[0] + accb_ref[1]], axis=1)
    h = jax.nn.relu(dv * acc + (2.0 * dv * dv) * xw_ref[...] + b_ref[...])
    xwn = jnp.dot(h, w_ref[...], preferred_element_type=jnp.float32)
    xwn_ref[...] = xwn
    yn = xwn * dv
    y0_ref[...] = yn[:, :FS]
    y1_ref[...] = yn[:, FS:]

  return pl.pallas_call(
      body,
      grid=(_GRID,),
      in_specs=[
          pl.BlockSpec((NC, ROW_BLK, FS), lambda i: (0, i, 0)),
          pl.BlockSpec((NC, ROW_BLK, FS), lambda i: (0, i, 0)),
          pl.BlockSpec((ROW_BLK, HH), lambda i: (i, 0)),
          pl.BlockSpec((ROW_BLK, 1), lambda i: (i, 0)),
          pl.BlockSpec((1, HH), lambda i: (0, 0)),
          pl.BlockSpec((HH, HH), lambda i: (0, 0)),
      ],
      out_specs=[
          pl.BlockSpec((ROW_BLK, HH), lambda i: (i, 0)),
          pl.BlockSpec((ROW_BLK, FS), lambda i: (i, 0)),
          pl.BlockSpec((ROW_BLK, FS), lambda i: (i, 0)),
      ],
      out_shape=[
          jax.ShapeDtypeStruct((N_PAD, HH), jnp.float32),
          jax.ShapeDtypeStruct((N_PAD, FS), jnp.float32),
          jax.ShapeDtypeStruct((N_PAD, FS), jnp.float32),
      ],
  )


def _tc_final():
  def body(acca_ref, accb_ref, xw_ref, dinv_ref, b_ref, batch_ref,
           ps_ref, cnt_ref, pool_ref):
    i = pl.program_id(0)
    dv = dinv_ref[...]
    acc = jnp.concatenate([acca_ref[0] + acca_ref[1],
                           accb_ref[0] + accb_ref[1]], axis=1)
    h = jax.nn.relu(dv * acc + (2.0 * dv * dv) * xw_ref[...] + b_ref[...])
    gid = lax.broadcasted_iota(jnp.int32, (ROW_BLK, GG), 1).astype(jnp.float32)
    oh = jnp.where(gid == batch_ref[...], 1.0, 0.0)

    @pl.when(i == 0)
    def _():
      ps_ref[...] = jnp.zeros_like(ps_ref)
      cnt_ref[...] = jnp.zeros_like(cnt_ref)

    ps_ref[...] += lax.dot_general(oh, h, (((0,), (0,)), ((), ())),
                                   preferred_element_type=jnp.float32)
    cnt_ref[...] += lax.dot_general(oh, jnp.ones((ROW_BLK, 1), jnp.float32),
                                    (((0,), (0,)), ((), ())),
                                    preferred_element_type=jnp.float32)

    @pl.when(i == _GRID - 1)
    def _():
      pool_ref[...] = ps_ref[...] / jnp.maximum(cnt_ref[...], 1.0)

  return pl.pallas_call(
      body,
      grid=(_GRID,),
      in_specs=[
          pl.BlockSpec((NC, ROW_BLK, FS), lambda i: (0, i, 0)),
          pl.BlockSpec((NC, ROW_BLK, FS), lambda i: (0, i, 0)),
          pl.BlockSpec((ROW_BLK, HH), lambda i: (i, 0)),
          pl.BlockSpec((ROW_BLK, 1), lambda i: (i, 0)),
          pl.BlockSpec((1, HH), lambda i: (0, 0)),
          pl.BlockSpec((ROW_BLK, 1), lambda i: (i, 0)),
      ],
      out_specs=[
          pl.BlockSpec((GG, HH), lambda i: (0, 0)),
          pl.BlockSpec((GG, 1), lambda i: (0, 0)),
          pl.BlockSpec((GG, HH), lambda i: (0, 0)),
      ],
      out_shape=[
          jax.ShapeDtypeStruct((GG, HH), jnp.float32),
          jax.ShapeDtypeStruct((GG, 1), jnp.float32),
          jax.ShapeDtypeStruct((GG, HH), jnp.float32),
      ],
  )


@jax.jit
def _run(x, edge_index, batch, W1, b1, W2, b2):
  src = edge_index[0]
  dst = edge_index[1]
  e = src.shape[0]
  e_pad = e
  src2 = src.reshape(NW, e_pad // (NW * CHUNK), CHUNK)
  dst2 = dst.reshape(NW, e_pad // (NW * CHUNK), CHUNK)

  x_p = jnp.pad(x, ((0, N_PAD - NN), (0, 0)))
  batch_p = jnp.concatenate(
      [batch.astype(jnp.float32),
       jnp.full((N_PAD - NN,), float(GG), jnp.float32)]).reshape(N_PAD, 1)

  ones16 = jnp.ones((CHUNK, HW), jnp.float32)
  zeros16 = jnp.zeros((ROWS_PER_TILE, HW), jnp.float32)

  degp = _sc_hist(e_pad)(dst2, ones16, zeros16)

  prep = _tc_prep()
  mid = _tc_mid()
  fin = _tc_final()
  conv = _sc_conv(e_pad)

  xw1, y1a, y1b, dinv, _xsum, hc, hcl = prep(x_p, W1, degp)
  acc1a, acc1b = conv(y1a, y1b, src2, dst2)
  xw2, y2a, y2b = mid(acc1a, acc1b, xw1, dinv, b1.reshape(1, HH), W2)
  acc2a, acc2b = conv(y2a, y2b, src2, dst2)
  xw3, y3a, y3b = mid(acc2a, acc2b, xw2, dinv, b2.reshape(1, HH), W2)
  acc3a, acc3b = conv(y3a, y3b, src2, dst2)
  _ps, _cnt, pooled = fin(acc3a, acc3b, xw3, dinv, b2.reshape(1, HH), batch_p)

  handcrafted = jnp.concatenate([hc.reshape(DD), hcl.reshape(1)])
  return pooled, handcrafted


def kernel(x, edge_index, batch, W1, b1, W2, b2):
  return _run(x, edge_index, batch, W1, b1, W2, b2)


# speedup vs baseline: 1.1122x; 1.1122x over previous
import functools

import jax
import jax.numpy as jnp
from jax import lax
from jax.experimental import pallas as pl
from jax.experimental.pallas import tpu as pltpu
from jax.experimental.pallas import tpu_sc as plsc

NN = 10000
DD = 128
HH = 128
GG = 16

NC = 2
NS = 16
NW = NC * NS

CHUNK = 80
N_PAD = 10240
TRASH = NN
ROW_BLK = 1024
ROWS_PER_TILE = N_PAD // NS


HW = 8


def _sc_hist(e_pad):
  per_tile = e_pad // NW
  n_chunks = per_tile // CHUNK
  mesh = plsc.VectorSubcoreMesh(core_axis_name="c", subcore_axis_name="s")

  n_groups = n_chunks // 5

  @functools.partial(
      pl.kernel,
      mesh=mesh,
      out_type=jax.ShapeDtypeStruct((NC, N_PAD, HW), jnp.float32),
      compiler_params=pltpu.CompilerParams(use_tc_tiling_on_sc=False),
      scratch_types=[
          pltpu.VMEM((n_chunks, CHUNK), jnp.int32),
          pltpu.VMEM((CHUNK, HW), jnp.float32),
          pltpu.VMEM((ROWS_PER_TILE, HW), jnp.float32),
          pltpu.VMEM_SHARED((N_PAD, HW), jnp.float32),
          pltpu.SemaphoreType.DMA((5,)),
      ],
  )
  def hist(dst_hbm, ones_hbm, zeros_hbm, out_hbm,
           dsti_v, ones_v, big_v, acc_sh, ssem):
    c = lax.axis_index("c")
    s = lax.axis_index("s")
    wid = c * NS + s
    row0 = s * ROWS_PER_TILE
    pltpu.sync_copy(zeros_hbm, big_v)
    pltpu.sync_copy(big_v, acc_sh.at[pl.ds(row0, ROWS_PER_TILE)])
    pltpu.sync_copy(ones_hbm, ones_v)
    pltpu.sync_copy(dst_hbm.at[pl.ds(wid * n_chunks, n_chunks)], dsti_v)
    plsc.subcore_barrier()

    def scatter(j, b):
      return pltpu.make_async_copy(ones_v, acc_sh.at[dsti_v.at[j]],
                                   ssem.at[b])

    def grp(g, carry):
      for b in range(5):
        @pl.when(g > 0)
        def _():
          scatter(0, b).wait()
        scatter(g * 5 + b, b).start(add=True)
      return carry

    lax.fori_loop(0, n_groups, grp, 0)
    for b in range(5):
      scatter(0, b).wait()
    plsc.subcore_barrier()
    pltpu.sync_copy(acc_sh.at[pl.ds(row0, ROWS_PER_TILE)], big_v)
    pltpu.sync_copy(big_v, out_hbm.at[c, pl.ds(row0, ROWS_PER_TILE)])

  return hist


SPLIT = 2
FS = HH // SPLIT
ZCH = ROWS_PER_TILE // 4


NBUF = 5


def _sc_conv(e_pad):
  per_tile = e_pad // NW
  n_chunks = per_tile // CHUNK
  n_groups = n_chunks // NBUF
  mesh = plsc.VectorSubcoreMesh(core_axis_name="c", subcore_axis_name="s")

  @functools.partial(
      pl.kernel,
      mesh=mesh,
      out_type=jax.ShapeDtypeStruct((NC, N_PAD, HH), jnp.float32),
      compiler_params=pltpu.CompilerParams(use_tc_tiling_on_sc=False),
      scratch_types=[
          pltpu.VMEM((n_chunks, CHUNK), jnp.int32),
          pltpu.VMEM((n_chunks, CHUNK), jnp.int32),
          [pltpu.VMEM((CHUNK, FS), jnp.float32) for _ in range(NBUF)],
          pltpu.VMEM((ZCH, FS), jnp.float32),
          pltpu.VMEM((ZCH, FS), jnp.float32),
          pltpu.VMEM_SHARED((N_PAD, FS), jnp.float32),
          pltpu.SemaphoreType.DMA((NBUF,)),
          pltpu.SemaphoreType.DMA((NBUF,)),
      ],
  )
  def conv(y0, y1, src_hbm, dst_hbm, out_hbm,
           srci_v, dsti_v, rows, zero_v, rd_v, acc_sh, gsem, ssem):
    c = lax.axis_index("c")
    s = lax.axis_index("s")
    wid = c * NS + s
    row0 = s * ROWS_PER_TILE

    pltpu.sync_copy(src_hbm.at[pl.ds(wid * n_chunks, n_chunks)], srci_v)
    pltpu.sync_copy(dst_hbm.at[pl.ds(wid * n_chunks, n_chunks)], dsti_v)

    def zbody(r, carry):
      z = jnp.zeros((16,), jnp.float32)
      for g in range(FS // 16):
        zero_v[r, pl.ds(g * 16, 16)] = z
      return carry

    lax.fori_loop(0, ZCH, zbody, 0)

    def gather(j, b, y_hbm):
      return pltpu.make_async_copy(y_hbm.at[srci_v.at[j]], rows[b],
                                   gsem.at[b])

    def scatter(j, b):
      return pltpu.make_async_copy(rows[b], acc_sh.at[dsti_v.at[j]],
                                   ssem.at[b])

    for f, y_hbm in enumerate((y0, y1)):
      for q in range(ROWS_PER_TILE // ZCH):
        pltpu.sync_copy(zero_v, acc_sh.at[pl.ds(row0 + q * ZCH, ZCH)])
      plsc.subcore_barrier()

      for b in range(NBUF):
        gather(b, b, y_hbm).start()

      def grp(g, carry):
        for b in range(NBUF):
          j = g * NBUF + b
          gather(j, b, y_hbm).wait()
          desc = scatter(j, b)
          desc.start(add=True)

        @pl.when(g < n_groups - 1)
        def _():
          for b in range(NBUF):
            j = (g + 1) * NBUF + b
            scatter(j - NBUF, b).wait()
            gather(j, b, y_hbm).start()

        return carry

      lax.fori_loop(0, n_groups, grp, 0)
      for b in range(NBUF):
        scatter(0, b).wait()

      plsc.subcore_barrier()
      for q in range(ROWS_PER_TILE // ZCH):
        pltpu.sync_copy(acc_sh.at[pl.ds(row0 + q * ZCH, ZCH)], rd_v)
        pltpu.sync_copy(rd_v, out_hbm.at[c, pl.ds(row0 + q * ZCH, ZCH),
                                         pl.ds(f * FS, FS)])

  return conv


_GRID = N_PAD // ROW_BLK


def _tc_prep():
  def body(x_ref, w_ref, degp_ref, xw_ref, y0_ref, y1_ref, dinv_ref,
           xsum_ref, hc_ref, hcl_ref):
    i = pl.program_id(0)
    dinv = lax.rsqrt(degp_ref[0, :, 0:1] + degp_ref[1, :, 0:1] + 2.0)
    xw = jnp.dot(x_ref[...], w_ref[...], preferred_element_type=jnp.float32)
    xw_ref[...] = xw
    y = xw * dinv
    y0_ref[...] = y[:, :FS]
    y1_ref[...] = y[:, FS:]
    dinv_ref[...] = dinv

    @pl.when(i == 0)
    def _():
      xsum_ref[...] = jnp.zeros_like(xsum_ref)

    xsum_ref[...] += jnp.sum(x_ref[...], axis=0, keepdims=True)

    @pl.when(i == _GRID - 1)
    def _():
      gs = jnp.sum(xsum_ref[...])
      hc_ref[...] = xsum_ref[...] / gs
      hcl_ref[...] = jnp.full((1, 1), 0.0) + jnp.log(gs)

  return pl.pallas_call(
      body,
      grid=(_GRID,),
      in_specs=[
          pl.BlockSpec((ROW_BLK, DD), lambda i: (i, 0)),
          pl.BlockSpec((DD, HH), lambda i: (0, 0)),
          pl.BlockSpec((NC, ROW_BLK, HW), lambda i: (0, i, 0)),
      ],
      out_specs=[
          pl.BlockSpec((ROW_BLK, HH), lambda i: (i, 0)),
          pl.BlockSpec((ROW_BLK, FS), lambda i: (i, 0)),
          pl.BlockSpec((ROW_BLK, FS), lambda i: (i, 0)),
          pl.BlockSpec((ROW_BLK, 1), lambda i: (i, 0)),
          pl.BlockSpec((1, DD), lambda i: (0, 0)),
          pl.BlockSpec((1, DD), lambda i: (0, 0)),
          pl.BlockSpec((1, 1), lambda i: (0, 0)),
      ],
      out_shape=[
          jax.ShapeDtypeStruct((N_PAD, HH), jnp.float32),
          jax.ShapeDtypeStruct((N_PAD, FS), jnp.float32),
          jax.ShapeDtypeStruct((N_PAD, FS), jnp.float32),
          jax.ShapeDtypeStruct((N_PAD, 1), jnp.float32),
          jax.ShapeDtypeStruct((1, DD), jnp.float32),
          jax.ShapeDtypeStruct((1, DD), jnp.float32),
          jax.ShapeDtypeStruct((1, 1), jnp.float32),
      ],
  )


def _tc_mid():
  def body(acc_ref, xw_ref, dinv_ref, b_ref, w_ref,
           xwn_ref, y0_ref, y1_ref):
    dv = dinv_ref[...]
    h = jax.nn.relu(dv * (acc_ref[0] + acc_ref[1])
                    + (2.0 * dv * dv) * xw_ref[...] + b_ref[...])
    xwn = jnp.dot(h, w_ref[...], preferred_element_type=jnp.float32)
    xwn_ref[...] = xwn
    yn = xwn * dv
    y0_ref[...] = yn[:, :FS]
    y1_ref[...] = yn[:, FS:]

  return pl.pallas_call(
      body,
      grid=(_GRID,),
      in_specs=[
          pl.BlockSpec((NC, ROW_BLK, HH), lambda i: (0, i, 0)),
          pl.BlockSpec((ROW_BLK, HH), lambda i: (i, 0)),
          pl.BlockSpec((ROW_BLK, 1), lambda i: (i, 0)),
          pl.BlockSpec((1, HH), lambda i: (0, 0)),
          pl.BlockSpec((HH, HH), lambda i: (0, 0)),
      ],
      out_specs=[
          pl.BlockSpec((ROW_BLK, HH), lambda i: (i, 0)),
          pl.BlockSpec((ROW_BLK, FS), lambda i: (i, 0)),
          pl.BlockSpec((ROW_BLK, FS), lambda i: (i, 0)),
      ],
      out_shape=[
          jax.ShapeDtypeStruct((N_PAD, HH), jnp.float32),
          jax.ShapeDtypeStruct((N_PAD, FS), jnp.float32),
          jax.ShapeDtypeStruct((N_PAD, FS), jnp.float32),
      ],
  )


def _tc_final():
  def body(acc_ref, xw_ref, dinv_ref, b_ref, batch_ref,
           ps_ref, cnt_ref, pool_ref):
    i = pl.program_id(0)
    dv = dinv_ref[...]
    h = jax.nn.relu(dv * (acc_ref[0] + acc_ref[1])
                    + (2.0 * dv * dv) * xw_ref[...] + b_ref[...])
    gid = lax.broadcasted_iota(jnp.int32, (ROW_BLK, GG), 1).astype(jnp.float32)
    oh = jnp.where(gid == batch_ref[...], 1.0, 0.0)

    @pl.when(i == 0)
    def _():
      ps_ref[...] = jnp.zeros_like(ps_ref)
      cnt_ref[...] = jnp.zeros_like(cnt_ref)

    ps_ref[...] += lax.dot_general(oh, h, (((0,), (0,)), ((), ())),
                                   preferred_element_type=jnp.float32)
    cnt_ref[...] += lax.dot_general(oh, jnp.ones((ROW_BLK, 1), jnp.float32),
                                    (((0,), (0,)), ((), ())),
                                    preferred_element_type=jnp.float32)

    @pl.when(i == _GRID - 1)
    def _():
      pool_ref[...] = ps_ref[...] / jnp.maximum(cnt_ref[...], 1.0)

  return pl.pallas_call(
      body,
      grid=(_GRID,),
      in_specs=[
          pl.BlockSpec((NC, ROW_BLK, HH), lambda i: (0, i, 0)),
          pl.BlockSpec((ROW_BLK, HH), lambda i: (i, 0)),
          pl.BlockSpec((ROW_BLK, 1), lambda i: (i, 0)),
          pl.BlockSpec((1, HH), lambda i: (0, 0)),
          pl.BlockSpec((ROW_BLK, 1), lambda i: (i, 0)),
      ],
      out_specs=[
          pl.BlockSpec((GG, HH), lambda i: (0, 0)),
          pl.BlockSpec((GG, 1), lambda i: (0, 0)),
          pl.BlockSpec((GG, HH), lambda i: (0, 0)),
      ],
      out_shape=[
          jax.ShapeDtypeStruct((GG, HH), jnp.float32),
          jax.ShapeDtypeStruct((GG, 1), jnp.float32),
          jax.ShapeDtypeStruct((GG, HH), jnp.float32),
      ],
  )


@jax.jit
def _run(x, edge_index, batch, W1, b1, W2, b2):
  src = edge_index[0]
  dst = edge_index[1]
  e = src.shape[0]
  e_pad = e
  src2 = src.reshape(e_pad // CHUNK, CHUNK)
  dst2 = dst.reshape(e_pad // CHUNK, CHUNK)

  x_p = jnp.pad(x, ((0, N_PAD - NN), (0, 0)))
  batch_p = jnp.concatenate(
      [batch.astype(jnp.float32),
       jnp.full((N_PAD - NN,), float(GG), jnp.float32)]).reshape(N_PAD, 1)

  ones16 = jnp.ones((CHUNK, HW), jnp.float32)
  zeros16 = jnp.zeros((ROWS_PER_TILE, HW), jnp.float32)

  degp = _sc_hist(e_pad)(dst2, ones16, zeros16)

  prep = _tc_prep()
  mid = _tc_mid()
  fin = _tc_final()
  conv = _sc_conv(e_pad)

  xw1, y1a, y1b, dinv, _xsum, hc, hcl = prep(x_p, W1, degp)
  acc1 = conv(y1a, y1b, src2, dst2)
  xw2, y2a, y2b = mid(acc1, xw1, dinv, b1.reshape(1, HH), W2)
  acc2 = conv(y2a, y2b, src2, dst2)
  xw3, y3a, y3b = mid(acc2, xw2, dinv, b2.reshape(1, HH), W2)
  acc3 = conv(y3a, y3b, src2, dst2)
  _ps, _cnt, pooled = fin(acc3, xw3, dinv, b2.reshape(1, HH), batch_p)

  handcrafted = jnp.concatenate([hc.reshape(DD), hcl.reshape(1)])
  return pooled, handcrafted


def kernel(x, edge_index, batch, W1, b1, W2, b2):
  return _run(x, edge_index, batch, W1, b1, W2, b2)
